# Initial kernel scaffold; baseline (speedup 1.0000x reference)
import jax
import jax.numpy as jnp
from jax.experimental import pallas as pl


def kernel(feature_map, region_masks):
    raise NotImplementedError("write your pallas kernel here")


# speedup vs baseline: 36.0470x; 36.0470x over previous
import numpy as np
import jax
import jax.numpy as jnp
from jax import lax
from jax.experimental import pallas as pl

_P = 1024
_G = 32


def _build_bilinear_matrix():
    g = _G
    c = np.arange(g * g)
    i = c // g
    j = c % g
    y = i.astype(np.float64) / g * (g - 1)
    x = j.astype(np.float64) / g * (g - 1)
    y0 = np.clip(np.floor(y).astype(np.int64), 0, g - 1)
    x0 = np.clip(np.floor(x).astype(np.int64), 0, g - 1)
    y1 = np.clip(y0 + 1, 0, g - 1)
    x1 = np.clip(x0 + 1, 0, g - 1)
    wy = y - y0
    wx = x - x0
    A = np.zeros((g * g, g * g), dtype=np.float64)
    np.add.at(A, (c, y0 * g + x0), (1 - wy) * (1 - wx))
    np.add.at(A, (c, y0 * g + x1), (1 - wy) * wx)
    np.add.at(A, (c, y1 * g + x0), wy * (1 - wx))
    np.add.at(A, (c, y1 * g + x1), wy * wx)
    return A.astype(np.float32)


_A_CONST = _build_bilinear_matrix()


def _region_pool_kernel(mask_ref, fmap_ref, a_ref, out_ref):
    g = _G
    m = (mask_ref[0, 0] != 0).astype(jnp.bfloat16)
    p_i = lax.broadcasted_iota(jnp.int32, (512, g), 0)
    j_i = lax.broadcasted_iota(jnp.int32, (512, g), 1)
    U = (p_i // 16 == j_i).astype(jnp.bfloat16)
    colred = jnp.dot(m, U, preferred_element_type=jnp.float32)
    i_i = lax.broadcasted_iota(jnp.int32, (g, 512), 0)
    q_i = lax.broadcasted_iota(jnp.int32, (g, 512), 1)
    Ut = (q_i // 16 == i_i).astype(jnp.float32)
    cnt = jnp.dot(Ut, colred, preferred_element_type=jnp.float32)
    occ = cnt > 0.5

    ri = lax.broadcasted_iota(jnp.int32, (g, g), 0)
    ci = lax.broadcasted_iota(jnp.int32, (g, g), 1)
    nraw = jnp.sum(occ.astype(jnp.float32))
    occ = occ | ((ri == 0) & (ci == 0) & (nraw < 0.5))
    o = occ.astype(jnp.float32)

    TRIU = (ri <= ci).astype(jnp.float32)
    SL = (ri > ci).astype(jnp.float32)
    crow = jnp.dot(o, TRIU, preferred_element_type=jnp.float32)
    rowsum = jnp.sum(o, axis=1, keepdims=True)
    prefix = jnp.dot(SL, rowsum, preferred_element_type=jnp.float32)
    rank = prefix + crow - 1.0

    n = jnp.sum(o)
    qd = jnp.floor(float(_P) / n)
    rem = float(_P) - qd * n
    w = o * (qd + (rank < rem).astype(jnp.float32))

    y_i = lax.broadcasted_iota(jnp.int32, (g, g * g), 0)
    p_i2 = lax.broadcasted_iota(jnp.int32, (g, g * g), 1)
    Rep = (p_i2 // g == y_i).astype(jnp.float32)
    D = (p_i2 % g == y_i).astype(jnp.float32)
    wb = lax.dot_general(w, Rep, (((0,), (0,)), ((), ())),
                         preferred_element_type=jnp.float32)
    ones_row = jnp.full((1, g), 1.0, dtype=jnp.float32)
    wflat = jnp.dot(ones_row, wb * D,
                    preferred_element_type=jnp.float32)

    vf = jnp.dot(wflat, a_ref[...], preferred_element_type=jnp.float32)
    out = jnp.dot(vf, fmap_ref[0], preferred_element_type=jnp.float32)
    out_ref[0, 0] = out * (1.0 / float(_P))


def _make_call(B, R, H, W, HW, C, interpret=False):
    return pl.pallas_call(
        _region_pool_kernel,
        grid=(B, R),
        in_specs=[
            pl.BlockSpec((1, 1, H, W), lambda b, r: (b, r, 0, 0)),
            pl.BlockSpec((1, HW, C), lambda b, r: (b, 0, 0)),
            pl.BlockSpec((_G * _G, _G * _G), lambda b, r: (0, 0)),
        ],
        out_specs=pl.BlockSpec((1, 1, 1, C), lambda b, r: (b, r, 0, 0)),
        out_shape=jax.ShapeDtypeStruct((B, R, 1, C), jnp.float32),
        interpret=interpret,
    )


def kernel(feature_map, region_masks):
    B, HW, C = feature_map.shape
    _, R, H, W = region_masks.shape
    call = _make_call(B, R, H, W, HW, C)
    return call(region_masks, feature_map, jnp.asarray(_A_CONST))


# speedup vs baseline: 58.8768x; 1.6333x over previous
import numpy as np
import jax
import jax.numpy as jnp
from jax import lax
from jax.experimental import pallas as pl
from jax.experimental.pallas import tpu as pltpu

_P = 1024
_G = 32


def _build_bilinear_matrix():
    g = _G
    c = np.arange(g * g)
    i = c // g
    j = c % g
    y = i.astype(np.float64) / g * (g - 1)
    x = j.astype(np.float64) / g * (g - 1)
    y0 = np.clip(np.floor(y).astype(np.int64), 0, g - 1)
    x0 = np.clip(np.floor(x).astype(np.int64), 0, g - 1)
    y1 = np.clip(y0 + 1, 0, g - 1)
    x1 = np.clip(x0 + 1, 0, g - 1)
    wy = y - y0
    wx = x - x0
    A = np.zeros((g * g, g * g), dtype=np.float64)
    np.add.at(A, (c, y0 * g + x0), (1 - wy) * (1 - wx))
    np.add.at(A, (c, y0 * g + x1), (1 - wy) * wx)
    np.add.at(A, (c, y1 * g + x0), wy * (1 - wx))
    np.add.at(A, (c, y1 * g + x1), wy * wx)
    return A.astype(np.float32)


def _build_constants(H):
    g = _G
    bh = H // g
    p = np.arange(H)
    U = (p[:, None] // bh == np.arange(g)[None, :]).astype(np.float32)
    a = np.arange(g)
    TRIU = (a[:, None] <= a[None, :]).astype(np.float32)
    SL = (a[:, None] > a[None, :]).astype(np.float32)
    q = np.arange(g * g)
    Rep = (q[None, :] // g == a[:, None]).astype(np.float32)
    D = (q[None, :] % g == a[:, None]).astype(np.float32)
    return (
        jnp.asarray(U, dtype=jnp.bfloat16),
        jnp.asarray(U.T, dtype=jnp.float32),
        jnp.asarray(TRIU), jnp.asarray(SL),
        jnp.asarray(Rep), jnp.asarray(D),
        jnp.asarray(_build_bilinear_matrix()),
    )


def _region_pool_kernel(mask_ref, fmap_ref, u_ref, ut_ref, triu_ref, sl_ref,
                        rep_ref, d_ref, a_ref, out_ref, wall_ref):
    g = _G
    R = wall_ref.shape[0]
    r = pl.program_id(1)

    m = (mask_ref[0, 0] != 0).astype(jnp.bfloat16)
    colred = jnp.dot(m, u_ref[...], preferred_element_type=jnp.float32)
    cnt = jnp.dot(ut_ref[...], colred, preferred_element_type=jnp.float32)
    occ = cnt > 0.5

    ri = lax.broadcasted_iota(jnp.int32, (g, g), 0)
    ci = lax.broadcasted_iota(jnp.int32, (g, g), 1)
    nraw = jnp.sum(occ.astype(jnp.float32))
    occ = occ | ((ri == 0) & (ci == 0) & (nraw < 0.5))
    o = occ.astype(jnp.float32)

    crow = jnp.dot(o, triu_ref[...], preferred_element_type=jnp.float32)
    rowsum = jnp.sum(o, axis=1, keepdims=True)
    prefix = jnp.dot(sl_ref[...], rowsum, preferred_element_type=jnp.float32)
    rank = prefix + crow - 1.0

    n = jnp.sum(o)
    qd = jnp.floor(float(_P) / n)
    rem = float(_P) - qd * n
    w = o * (qd + (rank < rem).astype(jnp.float32))

    wb = lax.dot_general(w, rep_ref[...], (((0,), (0,)), ((), ())),
                         preferred_element_type=jnp.float32)
    wflat = jnp.sum(wb * d_ref[...], axis=0, keepdims=True)
    wall_ref[pl.ds(r, 1), :] = wflat

    @pl.when(r == R - 1)
    def _():
        vall = jnp.dot(wall_ref[...], a_ref[...],
                       preferred_element_type=jnp.float32)
        out = jnp.dot(vall, fmap_ref[0],
                      preferred_element_type=jnp.float32)
        out_ref[0, :, 0, :] = out * (1.0 / float(_P))


def _make_call(B, R, H, W, HW, C, interpret=False):
    full = lambda shape: pl.BlockSpec(shape, lambda b, r: (0,) * len(shape))
    return pl.pallas_call(
        _region_pool_kernel,
        grid=(B, R),
        in_specs=[
            pl.BlockSpec((1, 1, H, W), lambda b, r: (b, r, 0, 0)),
            pl.BlockSpec((1, HW, C), lambda b, r: (b, 0, 0)),
            full((H, _G)),
            full((_G, H)),
            full((_G, _G)),
            full((_G, _G)),
            full((_G, _G * _G)),
            full((_G, _G * _G)),
            full((_G * _G, _G * _G)),
        ],
        out_specs=pl.BlockSpec((1, R, 1, C), lambda b, r: (b, 0, 0, 0)),
        out_shape=jax.ShapeDtypeStruct((B, R, 1, C), jnp.float32),
        scratch_shapes=[pltpu.VMEM((R, _G * _G), jnp.float32)],
        interpret=interpret,
    )


def kernel(feature_map, region_masks):
    B, HW, C = feature_map.shape
    _, R, H, W = region_masks.shape
    consts = _build_constants(H)
    call = _make_call(B, R, H, W, HW, C)
    return call(region_masks, feature_map, *consts)
